# Initial kernel scaffold; baseline (speedup 1.0000x reference)
import jax
import jax.numpy as jnp
from jax.experimental import pallas as pl


def kernel(inp, edgidx, h, Wl, Wr, b):
    raise NotImplementedError("write your pallas kernel here")


# speedup vs baseline: 4.0488x; 4.0488x over previous
import functools

import jax
import jax.numpy as jnp
from jax import lax
from jax.experimental import pallas as pl
from jax.experimental.pallas import tpu as pltpu
from jax.experimental.pallas import tpu_sc as plsc

N = 10000
D = 128
NSUB = 16
NP = 10112
CHUNK = 128
NCH = 2560
EPAD = NCH * CHUNK
SB = 16
ACC_SLICES = ((0, 128), (128, 128), (256, 128), (384, 128), (512, 120))
BLK = 1000

f32 = jnp.float32


def _zero_buf(buf, rows, cols):
    z16 = jnp.zeros((16,), f32)

    def row(i, c):
        for j in range(cols // 16):
            buf[i, pl.ds(j * 16, 16)] = z16
        return c

    lax.fori_loop(0, rows, row, 0)


def _fill_ones(buf, rows, cols):
    o16 = jnp.ones((16,), f32)

    def row(i, c):
        for j in range(cols // 16):
            buf[i, pl.ds(j * 16, 16)] = o16
        return c

    lax.fori_loop(0, rows, row, 0)


def _zero_acc_slice(sid, zbuf, acc):
    arow = sid * (NP // NSUB)
    for off, sz in ACC_SLICES:
        pltpu.sync_copy(zbuf.at[pl.ds(0, sz)], acc.at[pl.ds(arow + off, sz)])


def _agg_edges(x_hbm, ecc, wbase, nblk, ib0, ib1, isem0, isem1,
               buf0, buf1, sem0, sem1, acc):
    ibufs = (ib0, ib1)
    isems = (isem0, isem1)
    pltpu.async_copy(ecc.at[pl.ds(wbase, SB)], ib0, isem0)
    for blk in range(nblk):
        ib = ibufs[blk % 2]
        pltpu.make_async_copy(
            ecc.at[pl.ds(wbase + blk * SB, SB)], ib, isems[blk % 2]).wait()
        if blk + 1 < nblk:
            pltpu.async_copy(ecc.at[pl.ds(wbase + (blk + 1) * SB, SB)],
                             ibufs[(blk + 1) % 2], isems[(blk + 1) % 2])

        def body2(k, c):
            cp0 = pltpu.async_copy(x_hbm.at[ib.at[2 * k, 0]], buf0, sem0)
            cp1 = pltpu.async_copy(x_hbm.at[ib.at[2 * k + 1, 0]], buf1, sem1)
            cp0.wait()
            pltpu.sync_copy(buf0, acc.at[ib.at[2 * k, 1]], add=True)
            cp1.wait()
            pltpu.sync_copy(buf1, acc.at[ib.at[2 * k + 1, 1]], add=True)
            return c

        lax.fori_loop(0, SB // 2, body2, 0)


def _make_agg(dual):
    tpt = (NCH // NSUB) if dual else (NCH // (2 * NSUB))
    nblk = tpt // SB
    mesh = plsc.VectorSubcoreMesh(core_axis_name="c", subcore_axis_name="s")

    if dual:
        out_type = [jax.ShapeDtypeStruct((NP, D), f32),
                    jax.ShapeDtypeStruct((NP, D), f32)]
    else:
        out_type = [jax.ShapeDtypeStruct((2, NP, D), f32)]

    scratch = [
        pltpu.VMEM((SB, 2, CHUNK), jnp.int32),
        pltpu.VMEM((SB, 2, CHUNK), jnp.int32),
        pltpu.VMEM((CHUNK, D), f32),
        pltpu.VMEM((CHUNK, D), f32),
        pltpu.VMEM_SHARED((NP, D), f32),
        pltpu.SemaphoreType.DMA,
        pltpu.SemaphoreType.DMA,
        pltpu.SemaphoreType.DMA,
        pltpu.SemaphoreType.DMA,
    ]

    def body(*refs):
        if dual:
            x0, x1, ecc = refs[:3]
            out0, out1 = refs[3:5]
            refs = refs[5:]
        else:
            x0, ecc = refs[:2]
            x1 = x0
            out01 = refs[2]
            refs = refs[3:]
        ib0, ib1, buf0, buf1, acc, isem0, isem1, sem0, sem1 = refs

        cid = lax.axis_index("c")
        sid = lax.axis_index("s")

        _zero_buf(buf0, CHUNK, D)
        _zero_acc_slice(sid, buf0, acc)
        plsc.subcore_barrier()

        wbase = (sid if dual else cid * NSUB + sid) * tpt

        if dual:
            @pl.when(cid == 0)
            def _():
                _agg_edges(x0, ecc, wbase, nblk, ib0, ib1, isem0, isem1,
                           buf0, buf1, sem0, sem1, acc)

            @pl.when(cid == 1)
            def _():
                _agg_edges(x1, ecc, wbase, nblk, ib0, ib1, isem0, isem1,
                           buf0, buf1, sem0, sem1, acc)
        else:
            _agg_edges(x0, ecc, wbase, nblk, ib0, ib1, isem0, isem1,
                       buf0, buf1, sem0, sem1, acc)

        plsc.subcore_barrier()

        orow = sid * (NP // NSUB)
        osz = NP // NSUB

        if dual:
            @pl.when(cid == 0)
            def _():
                pltpu.sync_copy(acc.at[pl.ds(orow, osz)],
                                out0.at[pl.ds(orow, osz)])

            @pl.when(cid == 1)
            def _():
                pltpu.sync_copy(acc.at[pl.ds(orow, osz)],
                                out1.at[pl.ds(orow, osz)])
        else:
            pltpu.sync_copy(acc.at[pl.ds(orow, osz)],
                            out01.at[cid, pl.ds(orow, osz)])

    return functools.partial(pl.kernel, body, out_type=out_type, mesh=mesh,
                             scratch_types=scratch)()


def _make_deg():
    tpt = NCH // (2 * NSUB)
    mesh = plsc.VectorSubcoreMesh(core_axis_name="c", subcore_axis_name="s")
    out_type = [jax.ShapeDtypeStruct((2, NP, D), f32)]
    scratch = [
        pltpu.VMEM((tpt, 2, CHUNK), jnp.int32),
        pltpu.VMEM((CHUNK, D), f32),
        pltpu.VMEM_SHARED((NP, D), f32),
        pltpu.SemaphoreType.DMA,
    ]

    def body(ecc, out01, ib, sbuf, dacc, sem):
        cid = lax.axis_index("c")
        sid = lax.axis_index("s")

        _zero_buf(sbuf, CHUNK, D)
        _zero_acc_slice(sid, sbuf, dacc)
        _fill_ones(sbuf, CHUNK, D)
        plsc.subcore_barrier()

        wbase = (cid * NSUB + sid) * tpt
        pltpu.sync_copy(ecc.at[pl.ds(wbase, tpt)], ib)

        def chunk(j, c):
            pltpu.sync_copy(sbuf, dacc.at[ib.at[j, 1]], add=True)
            return c

        lax.fori_loop(0, tpt, chunk, 0)
        plsc.subcore_barrier()

        orow = sid * (NP // NSUB)
        osz = NP // NSUB
        pltpu.sync_copy(dacc.at[pl.ds(orow, osz)],
                        out01.at[cid, pl.ds(orow, osz)])

    return functools.partial(pl.kernel, body, out_type=out_type, mesh=mesh,
                             scratch_types=scratch)()


def _make_gate_a(split):
    def body(*refs):
        if split:
            mxa, mxb, mh, xin, hh, dega, degb, w, bb, z_o, rh_o, t1_o = refs
            mxv = mxa[...] + mxb[...]
        else:
            mxa, mh, xin, hh, dega, degb, w, bb, z_o, rh_o, t1_o = refs
            mxv = mxa[...]
        deg = dega[...][:, :1] + degb[...][:, :1]
        dinv = 1.0 / jnp.maximum(deg, 1.0)
        lhs = jnp.concatenate(
            [mxv * dinv, xin[...], mh[...] * dinv, hh[...]], axis=1)
        pre = jnp.dot(lhs, w[...], preferred_element_type=f32) + bb[...]
        z = jax.nn.sigmoid(pre[:, 0:D])
        r = jax.nn.sigmoid(pre[:, D:2 * D])
        z_o[...] = z
        rh_o[...] = r * hh[...]
        t1_o[...] = pre[:, 2 * D:3 * D]

    row = pl.BlockSpec((BLK, D), lambda i: (i, 0))
    deg_s = pl.BlockSpec((BLK, D), lambda i: (i, 0))
    w_s = pl.BlockSpec((4 * D, 3 * D), lambda i: (0, 0))
    b_s = pl.BlockSpec((1, 3 * D), lambda i: (0, 0))
    in_specs = ([row, row] if split else [row]) + [row, row, row, deg_s, deg_s,
                                                   w_s, b_s]
    return pl.pallas_call(
        body,
        grid=(N // BLK,),
        in_specs=in_specs,
        out_specs=[row, row, row],
        out_shape=[jax.ShapeDtypeStruct((N, D), f32)] * 3,
    )


def _make_gate_b(split):
    def body(*refs):
        if split:
            ma, mb, rh, t1, z, hh, dega, degb, w, ho = refs
            mv = ma[...] + mb[...]
        else:
            ma, rh, t1, z, hh, dega, degb, w, ho = refs
            mv = ma[...]
        deg = dega[...][:, :1] + degb[...][:, :1]
        dinv = 1.0 / jnp.maximum(deg, 1.0)
        lhs = jnp.concatenate([mv * dinv, rh[...]], axis=1)
        ht = jnp.tanh(t1[...] + jnp.dot(lhs, w[...], preferred_element_type=f32))
        zv = z[...]
        ho[...] = zv * hh[...] + (1.0 - zv) * ht

    row = pl.BlockSpec((BLK, D), lambda i: (i, 0))
    deg_s = pl.BlockSpec((BLK, D), lambda i: (i, 0))
    w_s = pl.BlockSpec((2 * D, D), lambda i: (0, 0))
    in_specs = ([row, row] if split else [row]) + [row, row, row, row,
                                                   deg_s, deg_s, w_s]
    return pl.pallas_call(
        body,
        grid=(N // BLK,),
        in_specs=in_specs,
        out_specs=row,
        out_shape=jax.ShapeDtypeStruct((N, D), f32),
    )


def kernel(inp, edgidx, h, Wl, Wr, b):
    src = edgidx[0]
    dst = edgidx[1]
    pad = EPAD - src.shape[0]
    srcc = jnp.concatenate(
        [src.astype(jnp.int32), jnp.zeros((pad,), jnp.int32)]).reshape(NCH, CHUNK)
    dstc = jnp.concatenate(
        [dst.astype(jnp.int32), jnp.full((pad,), N, jnp.int32)]).reshape(NCH, CHUNK)
    ecc = jnp.stack([srcc, dstc], axis=1)

    def w_a(i):
        z = jnp.zeros((D, D), f32)
        return jnp.concatenate([
            jnp.concatenate([Wl[i, 0], Wl[i, 2], Wl[i, 4]], axis=1),
            jnp.concatenate([Wr[i, 0], Wr[i, 2], Wr[i, 4]], axis=1),
            jnp.concatenate([Wl[i, 1], Wl[i, 3], z], axis=1),
            jnp.concatenate([Wr[i, 1], Wr[i, 3], z], axis=1),
        ], axis=0)

    def b_a(i):
        return jnp.concatenate(
            [b[i, 0] + b[i, 1], b[i, 2] + b[i, 3], b[i, 4] + b[i, 5]]
        ).reshape(1, 3 * D)

    def w_b(i):
        return jnp.concatenate([Wl[i, 5], Wr[i, 5]], axis=0)

    agg_dual = _make_agg(dual=True)
    agg_single = _make_agg(dual=False)
    deg_pass = _make_deg()
    gate_a = _make_gate_a(False)
    gate_a_split = _make_gate_a(True)
    gate_b = _make_gate_b(False)
    gate_b_split = _make_gate_b(True)

    dg = deg_pass(ecc)[0]
    dega, degb = dg[0], dg[1]

    mx0, mh0 = agg_dual(inp, h[0], ecc)
    z0, rh0, t10 = gate_a(mx0, mh0, inp, h[0], dega, degb, w_a(0), b_a(0))
    mrh0, mh1 = agg_dual(rh0, h[1], ecc)
    hout0 = gate_b(mrh0, rh0, t10, z0, h[0], dega, degb, w_b(0))

    mx1 = agg_single(hout0, ecc)[0]
    mx1a, mx1b = mx1[0], mx1[1]
    z1, rh1, t11 = gate_a_split(mx1a, mx1b, mh1, hout0, h[1], dega, degb,
                                w_a(1), b_a(1))
    mr1 = agg_single(rh1, ecc)[0]
    mr1a, mr1b = mr1[0], mr1[1]
    hout1 = gate_b_split(mr1a, mr1b, rh1, t11, z1, h[1], dega, degb, w_b(1))

    out = jnp.stack([hout0, hout1], axis=0)
    return (out, out)


# speedup vs baseline: 4.2667x; 1.0538x over previous
import functools

import jax
import jax.numpy as jnp
from jax import lax
from jax.experimental import pallas as pl
from jax.experimental.pallas import tpu as pltpu
from jax.experimental.pallas import tpu_sc as plsc

N = 10000
D = 128
NSUB = 16
NP = 10112
CHUNK = 128
NCH = 2560
EPAD = NCH * CHUNK
SB = 16
ACC_SLICES = ((0, 128), (128, 128), (256, 128), (384, 128), (512, 120))
BLK = 1000

f32 = jnp.float32


def _zero_buf(buf, rows, cols):
    z16 = jnp.zeros((16,), f32)

    def row(i, c):
        for j in range(cols // 16):
            buf[i, pl.ds(j * 16, 16)] = z16
        return c

    lax.fori_loop(0, rows, row, 0)


def _fill_ones(buf, rows, cols):
    o16 = jnp.ones((16,), f32)

    def row(i, c):
        for j in range(cols // 16):
            buf[i, pl.ds(j * 16, 16)] = o16
        return c

    lax.fori_loop(0, rows, row, 0)


def _zero_acc_slice(sid, zbuf, acc):
    arow = sid * (NP // NSUB)
    for off, sz in ACC_SLICES:
        pltpu.sync_copy(zbuf.at[pl.ds(0, sz)], acc.at[pl.ds(arow + off, sz)])


def _agg_edges(x_hbm, ecc, wbase, nblk, ib0, ib1, isem0, isem1,
               buf0, buf1, sem0, sem1, ssem0, ssem1, acc):
    ibufs = (ib0, ib1)
    isems = (isem0, isem1)
    pltpu.async_copy(ecc.at[pl.ds(wbase, SB)], ib0, isem0)
    for blk in range(nblk):
        ib = ibufs[blk % 2]
        pltpu.make_async_copy(
            ecc.at[pl.ds(wbase + blk * SB, SB)], ib, isems[blk % 2]).wait()
        if blk + 1 < nblk:
            pltpu.async_copy(ecc.at[pl.ds(wbase + (blk + 1) * SB, SB)],
                             ibufs[(blk + 1) % 2], isems[(blk + 1) % 2])

        pltpu.async_copy(x_hbm.at[ib.at[0, 0]], buf0, sem0)
        pltpu.async_copy(x_hbm.at[ib.at[1, 0]], buf1, sem1)

        def body2(k, c):
            j0, j1 = 2 * k, 2 * k + 1
            pltpu.make_async_copy(x_hbm.at[ib.at[j0, 0]], buf0, sem0).wait()
            pltpu.async_copy(buf0, acc.at[ib.at[j0, 1]], ssem0, add=True)
            pltpu.make_async_copy(x_hbm.at[ib.at[j1, 0]], buf1, sem1).wait()
            pltpu.async_copy(buf1, acc.at[ib.at[j1, 1]], ssem1, add=True)

            @pl.when(k < SB // 2 - 1)
            def _():
                pltpu.make_async_copy(buf0, acc.at[ib.at[j0, 1]],
                                      ssem0).wait()
                pltpu.async_copy(x_hbm.at[ib.at[j0 + 2, 0]], buf0, sem0)
                pltpu.make_async_copy(buf1, acc.at[ib.at[j1, 1]],
                                      ssem1).wait()
                pltpu.async_copy(x_hbm.at[ib.at[j1 + 2, 0]], buf1, sem1)

            return c

        lax.fori_loop(0, SB // 2, body2, 0)
        pltpu.make_async_copy(buf0, acc.at[ib.at[SB - 2, 1]], ssem0).wait()
        pltpu.make_async_copy(buf1, acc.at[ib.at[SB - 1, 1]], ssem1).wait()


def _make_agg(dual):
    tpt = (NCH // NSUB) if dual else (NCH // (2 * NSUB))
    nblk = tpt // SB
    mesh = plsc.VectorSubcoreMesh(core_axis_name="c", subcore_axis_name="s")

    if dual:
        out_type = [jax.ShapeDtypeStruct((NP, D), f32),
                    jax.ShapeDtypeStruct((NP, D), f32)]
    else:
        out_type = [jax.ShapeDtypeStruct((2, NP, D), f32)]

    scratch = [
        pltpu.VMEM((SB, 2, CHUNK), jnp.int32),
        pltpu.VMEM((SB, 2, CHUNK), jnp.int32),
        pltpu.VMEM((CHUNK, D), f32),
        pltpu.VMEM((CHUNK, D), f32),
        pltpu.VMEM_SHARED((NP, D), f32),
        pltpu.SemaphoreType.DMA,
        pltpu.SemaphoreType.DMA,
        pltpu.SemaphoreType.DMA,
        pltpu.SemaphoreType.DMA,
        pltpu.SemaphoreType.DMA,
        pltpu.SemaphoreType.DMA,
    ]

    def body(*refs):
        if dual:
            x0, x1, ecc = refs[:3]
            out0, out1 = refs[3:5]
            refs = refs[5:]
        else:
            x0, ecc = refs[:2]
            x1 = x0
            out01 = refs[2]
            refs = refs[3:]
        (ib0, ib1, buf0, buf1, acc,
         isem0, isem1, sem0, sem1, ssem0, ssem1) = refs

        cid = lax.axis_index("c")
        sid = lax.axis_index("s")

        _zero_buf(buf0, CHUNK, D)
        _zero_acc_slice(sid, buf0, acc)
        plsc.subcore_barrier()

        wbase = (sid if dual else cid * NSUB + sid) * tpt

        if dual:
            @pl.when(cid == 0)
            def _():
                _agg_edges(x0, ecc, wbase, nblk, ib0, ib1, isem0, isem1,
                           buf0, buf1, sem0, sem1, ssem0, ssem1, acc)

            @pl.when(cid == 1)
            def _():
                _agg_edges(x1, ecc, wbase, nblk, ib0, ib1, isem0, isem1,
                           buf0, buf1, sem0, sem1, ssem0, ssem1, acc)
        else:
            _agg_edges(x0, ecc, wbase, nblk, ib0, ib1, isem0, isem1,
                       buf0, buf1, sem0, sem1, ssem0, ssem1, acc)

        plsc.subcore_barrier()

        orow = sid * (NP // NSUB)
        osz = NP // NSUB

        if dual:
            @pl.when(cid == 0)
            def _():
                pltpu.sync_copy(acc.at[pl.ds(orow, osz)],
                                out0.at[pl.ds(orow, osz)])

            @pl.when(cid == 1)
            def _():
                pltpu.sync_copy(acc.at[pl.ds(orow, osz)],
                                out1.at[pl.ds(orow, osz)])
        else:
            pltpu.sync_copy(acc.at[pl.ds(orow, osz)],
                            out01.at[cid, pl.ds(orow, osz)])

    return functools.partial(pl.kernel, body, out_type=out_type, mesh=mesh,
                             scratch_types=scratch)()


def _make_deg():
    tpt = NCH // (2 * NSUB)
    mesh = plsc.VectorSubcoreMesh(core_axis_name="c", subcore_axis_name="s")
    out_type = [jax.ShapeDtypeStruct((2, NP, D), f32)]
    scratch = [
        pltpu.VMEM((tpt, 2, CHUNK), jnp.int32),
        pltpu.VMEM((CHUNK, D), f32),
        pltpu.VMEM_SHARED((NP, D), f32),
        pltpu.SemaphoreType.DMA,
    ]

    def body(ecc, out01, ib, sbuf, dacc, sem):
        cid = lax.axis_index("c")
        sid = lax.axis_index("s")

        _zero_buf(sbuf, CHUNK, D)
        _zero_acc_slice(sid, sbuf, dacc)
        _fill_ones(sbuf, CHUNK, D)
        plsc.subcore_barrier()

        wbase = (cid * NSUB + sid) * tpt
        pltpu.sync_copy(ecc.at[pl.ds(wbase, tpt)], ib)

        def chunk(j, c):
            pltpu.sync_copy(sbuf, dacc.at[ib.at[j, 1]], add=True)
            return c

        lax.fori_loop(0, tpt, chunk, 0)
        plsc.subcore_barrier()

        orow = sid * (NP // NSUB)
        osz = NP // NSUB
        pltpu.sync_copy(dacc.at[pl.ds(orow, osz)],
                        out01.at[cid, pl.ds(orow, osz)])

    return functools.partial(pl.kernel, body, out_type=out_type, mesh=mesh,
                             scratch_types=scratch)()


def _make_gate_a(split):
    def body(*refs):
        if split:
            mxa, mxb, mh, xin, hh, dega, degb, w, bb, z_o, rh_o, t1_o = refs
            mxv = mxa[...] + mxb[...]
        else:
            mxa, mh, xin, hh, dega, degb, w, bb, z_o, rh_o, t1_o = refs
            mxv = mxa[...]
        deg = dega[...][:, :1] + degb[...][:, :1]
        dinv = 1.0 / jnp.maximum(deg, 1.0)
        lhs = jnp.concatenate(
            [mxv * dinv, xin[...], mh[...] * dinv, hh[...]], axis=1)
        pre = jnp.dot(lhs, w[...], preferred_element_type=f32) + bb[...]
        z = jax.nn.sigmoid(pre[:, 0:D])
        r = jax.nn.sigmoid(pre[:, D:2 * D])
        z_o[...] = z
        rh_o[...] = r * hh[...]
        t1_o[...] = pre[:, 2 * D:3 * D]

    row = pl.BlockSpec((BLK, D), lambda i: (i, 0))
    deg_s = pl.BlockSpec((BLK, D), lambda i: (i, 0))
    w_s = pl.BlockSpec((4 * D, 3 * D), lambda i: (0, 0))
    b_s = pl.BlockSpec((1, 3 * D), lambda i: (0, 0))
    in_specs = ([row, row] if split else [row]) + [row, row, row, deg_s, deg_s,
                                                   w_s, b_s]
    return pl.pallas_call(
        body,
        grid=(N // BLK,),
        in_specs=in_specs,
        out_specs=[row, row, row],
        out_shape=[jax.ShapeDtypeStruct((N, D), f32)] * 3,
    )


def _make_gate_b(split):
    def body(*refs):
        if split:
            ma, mb, rh, t1, z, hh, dega, degb, w, ho = refs
            mv = ma[...] + mb[...]
        else:
            ma, rh, t1, z, hh, dega, degb, w, ho = refs
            mv = ma[...]
        deg = dega[...][:, :1] + degb[...][:, :1]
        dinv = 1.0 / jnp.maximum(deg, 1.0)
        lhs = jnp.concatenate([mv * dinv, rh[...]], axis=1)
        ht = jnp.tanh(t1[...] + jnp.dot(lhs, w[...], preferred_element_type=f32))
        zv = z[...]
        ho[...] = zv * hh[...] + (1.0 - zv) * ht

    row = pl.BlockSpec((BLK, D), lambda i: (i, 0))
    deg_s = pl.BlockSpec((BLK, D), lambda i: (i, 0))
    w_s = pl.BlockSpec((2 * D, D), lambda i: (0, 0))
    in_specs = ([row, row] if split else [row]) + [row, row, row, row,
                                                   deg_s, deg_s, w_s]
    return pl.pallas_call(
        body,
        grid=(N // BLK,),
        in_specs=in_specs,
        out_specs=row,
        out_shape=jax.ShapeDtypeStruct((N, D), f32),
    )


def kernel(inp, edgidx, h, Wl, Wr, b):
    src = edgidx[0]
    dst = edgidx[1]
    pad = EPAD - src.shape[0]
    srcc = jnp.concatenate(
        [src.astype(jnp.int32), jnp.zeros((pad,), jnp.int32)]).reshape(NCH, CHUNK)
    dstc = jnp.concatenate(
        [dst.astype(jnp.int32), jnp.full((pad,), N, jnp.int32)]).reshape(NCH, CHUNK)
    ecc = jnp.stack([srcc, dstc], axis=1)

    def w_a(i):
        z = jnp.zeros((D, D), f32)
        return jnp.concatenate([
            jnp.concatenate([Wl[i, 0], Wl[i, 2], Wl[i, 4]], axis=1),
            jnp.concatenate([Wr[i, 0], Wr[i, 2], Wr[i, 4]], axis=1),
            jnp.concatenate([Wl[i, 1], Wl[i, 3], z], axis=1),
            jnp.concatenate([Wr[i, 1], Wr[i, 3], z], axis=1),
        ], axis=0)

    def b_a(i):
        return jnp.concatenate(
            [b[i, 0] + b[i, 1], b[i, 2] + b[i, 3], b[i, 4] + b[i, 5]]
        ).reshape(1, 3 * D)

    def w_b(i):
        return jnp.concatenate([Wl[i, 5], Wr[i, 5]], axis=0)

    agg_dual = _make_agg(dual=True)
    agg_single = _make_agg(dual=False)
    deg_pass = _make_deg()
    gate_a = _make_gate_a(False)
    gate_a_split = _make_gate_a(True)
    gate_b = _make_gate_b(False)
    gate_b_split = _make_gate_b(True)

    dg = deg_pass(ecc)[0]
    dega, degb = dg[0], dg[1]

    mx0, mh0 = agg_dual(inp, h[0], ecc)
    z0, rh0, t10 = gate_a(mx0, mh0, inp, h[0], dega, degb, w_a(0), b_a(0))
    mrh0, mh1 = agg_dual(rh0, h[1], ecc)
    hout0 = gate_b(mrh0, rh0, t10, z0, h[0], dega, degb, w_b(0))

    mx1 = agg_single(hout0, ecc)[0]
    mx1a, mx1b = mx1[0], mx1[1]
    z1, rh1, t11 = gate_a_split(mx1a, mx1b, mh1, hout0, h[1], dega, degb,
                                w_a(1), b_a(1))
    mr1 = agg_single(rh1, ecc)[0]
    mr1a, mr1b = mr1[0], mr1[1]
    hout1 = gate_b_split(mr1a, mr1b, rh1, t11, z1, h[1], dega, degb, w_b(1))

    out = jnp.stack([hout0, hout1], axis=0)
    return (out, out)
